# Initial kernel scaffold; baseline (speedup 1.0000x reference)
import jax
import jax.numpy as jnp
from jax.experimental import pallas as pl


def kernel(x, edge_index, W0, b0, W1, b1):
    raise NotImplementedError("write your pallas kernel here")


# speedup vs baseline: 9.1880x; 9.1880x over previous
import jax
import jax.numpy as jnp
from jax import lax
from jax.experimental import pallas as pl
from jax.experimental.pallas import tpu as pltpu
from jax.experimental.pallas import tpu_sc as plsc

N_NODES = 10000
NFEAT = 128
HALF = 64
N_POWERS = 10

NC = 2
NS = 16
CHUNK = 128
NBUF = 3

NPAD = 10240
ROWS_PER_TILE = NPAD // NS
ROW_CHUNKS = ROWS_PER_TILE // CHUNK
DUMMY_ROW = NPAD - 1

E_TOT = 320000 + N_NODES
N_SLICES = 2 * NS
EDGE_GROUPS = 27
CHUNKS_PER_SLICE = EDGE_GROUPS * NBUF
E_PAD = N_SLICES * CHUNKS_PER_SLICE * CHUNK


def _mlp_body(x_ref, w0_ref, b0_ref, w1_ref, b1_ref,
              i0_ref, i1_ref, o0_ref, o1_ref):
    h = jnp.maximum(
        jnp.dot(x_ref[...], w0_ref[...], preferred_element_type=jnp.float32)
        + b0_ref[...], 0.0)
    o = (jnp.dot(h, w1_ref[...], preferred_element_type=jnp.float32)
         + b1_ref[...])
    i0_ref[...] = o[:, :HALF]
    i1_ref[...] = o[:, HALF:]
    o0_ref[...] = 0.5 * o[:, :HALF]
    o1_ref[...] = 0.5 * o[:, HALF:]


def _mlp(x_pad, W0, b0, W1, b1):
    blk = 1024
    grid = NPAD // blk
    outs = [jax.ShapeDtypeStruct((NPAD, HALF), jnp.float32)] * 4
    full = lambda i: (0, 0)
    return pl.pallas_call(
        _mlp_body,
        grid=(grid,),
        in_specs=[
            pl.BlockSpec((blk, NFEAT), lambda i: (i, 0)),
            pl.BlockSpec((NFEAT, NFEAT), full),
            pl.BlockSpec((1, NFEAT), full),
            pl.BlockSpec((NFEAT, NFEAT), full),
            pl.BlockSpec((1, NFEAT), full),
        ],
        out_specs=[pl.BlockSpec((blk, HALF), lambda i: (i, 0))] * 4,
        out_shape=outs,
    )(x_pad, W0, b0.reshape(1, NFEAT), W1, b1.reshape(1, NFEAT))


def _sc_body(idxm, invcat, o2cat, ones_h, zeros_h,
             res,
             idx_v, bufs, res_v, o2_v, zero_v, ones_v, deg_v,
             agg_sh, deg_sh,
             isem0, isem1, isem2, gsem0, gsem1, gsem2,
             ssem0, ssem1, ssem2):
    c = lax.axis_index("c")
    s = lax.axis_index("s")
    isems = (isem0, isem1, isem2)
    gsems = (gsem0, gsem1, gsem2)
    ssems = (ssem0, ssem1, ssem2)
    sl0 = c * N_SLICES + s
    sl1 = c * N_SLICES + s + NS

    pltpu.sync_copy(ones_h, ones_v)
    pltpu.sync_copy(zeros_h, zero_v)
    for k in range(ROW_CHUNKS):
        base_l = s * ROWS_PER_TILE + k * CHUNK
        base_g = c * NPAD + base_l
        pltpu.sync_copy(zero_v, agg_sh.at[pl.ds(base_l, CHUNK)])
        pltpu.sync_copy(zero_v.at[:, pl.ds(0, 16)],
                        deg_sh.at[pl.ds(base_l, CHUNK)])
        pltpu.sync_copy(invcat.at[pl.ds(base_g, CHUNK)], res_v)
        pltpu.sync_copy(res_v, res.at[pl.ds(base_g, CHUNK)])
    plsc.subcore_barrier()

    def deg_chunk(j, sl):
        pltpu.sync_copy(idxm.at[sl, j], idx_v.at[0])
        pltpu.sync_copy(ones_v, deg_sh.at[idx_v.at[0, 1]], add=True)
        return sl

    lax.fori_loop(0, CHUNKS_PER_SLICE, deg_chunk, sl0)
    lax.fori_loop(0, CHUNKS_PER_SLICE, deg_chunk, sl1)
    plsc.subcore_barrier()

    def power_iter(_, carry):
        for sl in (sl0, sl1):
            def group(g, slc):
                idm = [pltpu.async_copy(idxm.at[slc, g * NBUF + b],
                                        idx_v.at[b], isems[b])
                       for b in range(NBUF)]
                gd = []
                for b in range(NBUF):
                    idm[b].wait()
                    gd.append(pltpu.async_copy(
                        res.at[idx_v.at[b, 0]], bufs.at[b], gsems[b]))
                sd = []
                for b in range(NBUF):
                    gd[b].wait()
                    sd.append(pltpu.async_copy(
                        bufs.at[b], agg_sh.at[idx_v.at[b, 1]],
                        ssems[b], add=True))
                for d in sd:
                    d.wait()
                return slc

            lax.fori_loop(0, EDGE_GROUPS, group, sl)
        plsc.subcore_barrier()

        for k in range(ROW_CHUNKS):
            base_l = s * ROWS_PER_TILE + k * CHUNK
            base_g = c * NPAD + base_l
            pltpu.sync_copy(agg_sh.at[pl.ds(base_l, CHUNK)], res_v)
            pltpu.sync_copy(deg_sh.at[pl.ds(base_l, CHUNK)], deg_v)
            pltpu.sync_copy(o2cat.at[pl.ds(base_g, CHUNK)], o2_v)

            def rbody(r, kk):
                wv = 0.5 / deg_v[r, :]
                for f in range(HALF // 16):
                    sf = pl.ds(f * 16, 16)
                    res_v[r, sf] = wv * res_v[r, sf] + o2_v[r, sf]
                return kk

            lax.fori_loop(0, CHUNK, rbody, k)
            pltpu.sync_copy(res_v, res.at[pl.ds(base_g, CHUNK)])
            pltpu.sync_copy(zero_v, agg_sh.at[pl.ds(base_l, CHUNK)])
        plsc.subcore_barrier()
        return carry

    lax.fori_loop(0, N_POWERS, power_iter, 0)


def _propagate(idxm, invcat, o2cat):
    ones_h = jnp.ones((CHUNK, 16), jnp.float32)
    zeros_h = jnp.zeros((CHUNK, HALF), jnp.float32)
    mesh = plsc.VectorSubcoreMesh(core_axis_name="c", subcore_axis_name="s")
    kern = pl.kernel(
        _sc_body,
        out_type=jax.ShapeDtypeStruct((NC * NPAD, HALF), jnp.float32),
        mesh=mesh,
        scratch_types=[
            pltpu.VMEM((NBUF, 2, CHUNK), jnp.int32),
            pltpu.VMEM((NBUF, CHUNK, HALF), jnp.float32),
            pltpu.VMEM((CHUNK, HALF), jnp.float32),
            pltpu.VMEM((CHUNK, HALF), jnp.float32),
            pltpu.VMEM((CHUNK, HALF), jnp.float32),
            pltpu.VMEM((CHUNK, 16), jnp.float32),
            pltpu.VMEM((CHUNK, 16), jnp.float32),
            pltpu.VMEM_SHARED((NPAD, HALF), jnp.float32),
            pltpu.VMEM_SHARED((NPAD, 16), jnp.float32),
            pltpu.SemaphoreType.DMA,
            pltpu.SemaphoreType.DMA,
            pltpu.SemaphoreType.DMA,
            pltpu.SemaphoreType.DMA,
            pltpu.SemaphoreType.DMA,
            pltpu.SemaphoreType.DMA,
            pltpu.SemaphoreType.DMA,
            pltpu.SemaphoreType.DMA,
            pltpu.SemaphoreType.DMA,
        ],
        compiler_params=pltpu.CompilerParams(use_tc_tiling_on_sc=False),
    )
    return kern(idxm, invcat, o2cat, ones_h, zeros_h)


def kernel(x, edge_index, W0, b0, W1, b1):
    x_pad = jnp.pad(x, ((0, NPAD - N_NODES), (0, 0)))
    i0, i1, o0, o1 = _mlp(x_pad, W0, b0, W1, b1)
    invcat = jnp.concatenate([i0, i1], axis=0)
    o2cat = jnp.concatenate([o0, o1], axis=0)

    loop = jnp.arange(N_NODES, dtype=jnp.int32)
    row = jnp.concatenate([edge_index[0], loop])
    col = jnp.concatenate([edge_index[1], loop])
    rowm = jnp.pad(row, (0, E_PAD - E_TOT),
                   constant_values=DUMMY_ROW).reshape(
                       N_SLICES, CHUNKS_PER_SLICE, 1, CHUNK)
    colm = jnp.pad(col, (0, E_PAD - E_TOT), constant_values=0).reshape(
        N_SLICES, CHUNKS_PER_SLICE, 1, CHUNK)
    half0 = jnp.concatenate([colm, rowm], axis=2)
    half1 = jnp.concatenate([colm + NPAD, rowm], axis=2)
    idxm = jnp.concatenate([half0, half1], axis=0)

    res = _propagate(idxm, invcat, o2cat)
    return jnp.concatenate(
        [res[:N_NODES], res[NPAD:NPAD + N_NODES]], axis=1)


# speedup vs baseline: 11.0718x; 1.2050x over previous
import jax
import jax.numpy as jnp
from jax import lax
from jax.experimental import pallas as pl
from jax.experimental.pallas import tpu as pltpu
from jax.experimental.pallas import tpu_sc as plsc

N_NODES = 10000
NFEAT = 128
HALF = 64
N_POWERS = 10

NC = 2
NS = 16
CHUNK = 128
RING = 3

NPAD = 10240
ROWS_PER_TILE = NPAD // NS
ROW_CHUNKS = ROWS_PER_TILE // CHUNK
DUMMY_ROW = NPAD - 1

E_TOT = 320000 + N_NODES
N_SLICES = NS
EDGE_ROUNDS = 56
CHUNKS_PER_SLICE = EDGE_ROUNDS * RING
E_PAD = N_SLICES * CHUNKS_PER_SLICE * CHUNK


def _mlp_body(x_ref, w0_ref, b0_ref, w1_ref, b1_ref,
              i0_ref, i1_ref, o0_ref, o1_ref):
    h = jnp.maximum(
        jnp.dot(x_ref[...], w0_ref[...], preferred_element_type=jnp.float32)
        + b0_ref[...], 0.0)
    o = (jnp.dot(h, w1_ref[...], preferred_element_type=jnp.float32)
         + b1_ref[...])
    i0_ref[...] = o[:, :HALF]
    i1_ref[...] = o[:, HALF:]
    o0_ref[...] = 0.5 * o[:, :HALF]
    o1_ref[...] = 0.5 * o[:, HALF:]


def _mlp(x_pad, W0, b0, W1, b1):
    blk = 1024
    grid = NPAD // blk
    outs = [jax.ShapeDtypeStruct((NPAD, HALF), jnp.float32)] * 4
    full = lambda i: (0, 0)
    return pl.pallas_call(
        _mlp_body,
        grid=(grid,),
        in_specs=[
            pl.BlockSpec((blk, NFEAT), lambda i: (i, 0)),
            pl.BlockSpec((NFEAT, NFEAT), full),
            pl.BlockSpec((1, NFEAT), full),
            pl.BlockSpec((NFEAT, NFEAT), full),
            pl.BlockSpec((1, NFEAT), full),
        ],
        out_specs=[pl.BlockSpec((blk, HALF), lambda i: (i, 0))] * 4,
        out_shape=outs,
    )(x_pad, W0, b0.reshape(1, NFEAT), W1, b1.reshape(1, NFEAT))


def _sc_body(idxm, invcat, o2cat, ones_h, zeros_h,
             res,
             idx_v, bufs, zero_v, deg_v,
             t0_sh, t1_sh, deg_sh,
             isems, gsems, ssems, zsems):
    c = lax.axis_index("c")
    s = lax.axis_index("s")

    def lchunk(k):
        return pl.ds(s * ROWS_PER_TILE + k * CHUNK, CHUNK)

    def gchunk(k):
        return pl.ds(c * NPAD + s * ROWS_PER_TILE + k * CHUNK, CHUNK)

    dc = [pltpu.async_copy(ones_h, deg_v.at[1], isems[0]),
          pltpu.async_copy(zeros_h, zero_v, isems[1])]
    dc[1].wait()
    zd = []
    for k in range(ROW_CHUNKS):
        zd.append(pltpu.async_copy(zero_v, t1_sh.at[lchunk(k)], zsems[0]))
        zd.append(pltpu.async_copy(zero_v.at[:, pl.ds(0, 16)],
                                   deg_sh.at[lchunk(k)], zsems[1]))
    sd = {}
    for k in range(ROW_CHUNKS):
        b = k % RING
        if k >= RING:
            sd.pop(k - RING).wait()
        ld = pltpu.async_copy(invcat.at[gchunk(k)], bufs.at[b], gsems[b])
        ld.wait()
        sd[k] = pltpu.async_copy(bufs.at[b], t0_sh.at[lchunk(k)], ssems[b])
    dc[0].wait()
    for d in zd + list(sd.values()):
        d.wait()
    plsc.subcore_barrier()

    for b in range(RING):
        pltpu.async_copy(idxm.at[s, b], idx_v.at[b], isems[b])

    def deg_round(g, h, hn):
        for b in range(RING):
            @pl.when(g > 0)
            def _(b=b):
                pltpu.make_async_copy(deg_v.at[1],
                                      deg_sh.at[idx_v.at[hn + b, 1]],
                                      ssems[b]).wait()
            @pl.when(g + 1 < EDGE_ROUNDS)
            def _(b=b):
                pltpu.async_copy(idxm.at[s, (g + 1) * RING + b],
                                 idx_v.at[hn + b], isems[hn + b])
        for b in range(RING):
            pltpu.make_async_copy(idxm.at[s, g * RING + b],
                                  idx_v.at[h + b], isems[h + b]).wait()
            pltpu.async_copy(deg_v.at[1], deg_sh.at[idx_v.at[h + b, 1]],
                             ssems[b], add=True)

    def deg_pair(G, carry):
        deg_round(2 * G, 0, RING)
        deg_round(2 * G + 1, RING, 0)
        return carry

    lax.fori_loop(0, EDGE_ROUNDS // 2, deg_pair, 0)
    hl = ((EDGE_ROUNDS - 1) % 2) * RING
    for b in range(RING):
        pltpu.make_async_copy(deg_v.at[1], deg_sh.at[idx_v.at[hl + b, 1]],
                              ssems[b]).wait()
    plsc.subcore_barrier()

    for k in range(ROW_CHUNKS):
        pltpu.sync_copy(deg_sh.at[lchunk(k)], deg_v.at[0])

        def wbody(r, carry):
            deg_v[0, r, :] = 0.5 / deg_v[0, r, :]
            return carry

        lax.fori_loop(0, CHUNK, wbody, 0)
        pltpu.sync_copy(deg_v.at[0], deg_sh.at[lchunk(k)])
    plsc.subcore_barrier()

    def one_iter(src, dst):
        for b in range(RING):
            pltpu.async_copy(idxm.at[s, b], idx_v.at[b], isems[b])

        def edge_round(g, h, hn):
            for b in range(RING):
                @pl.when(g > 0)
                def _(b=b):
                    pltpu.make_async_copy(
                        bufs.at[b], dst.at[idx_v.at[hn + b, 1]],
                        ssems[b]).wait()
                @pl.when(g + 1 < EDGE_ROUNDS)
                def _(b=b):
                    pltpu.async_copy(idxm.at[s, (g + 1) * RING + b],
                                     idx_v.at[hn + b], isems[hn + b])
            for b in range(RING):
                pltpu.make_async_copy(idxm.at[s, g * RING + b],
                                      idx_v.at[h + b], isems[h + b]).wait()
                pltpu.async_copy(src.at[idx_v.at[h + b, 0]], bufs.at[b],
                                 gsems[b])
            for b in range(RING):
                pltpu.make_async_copy(src.at[idx_v.at[h + b, 0]], bufs.at[b],
                                      gsems[b]).wait()
                pltpu.async_copy(bufs.at[b], dst.at[idx_v.at[h + b, 1]],
                                 ssems[b], add=True)

        def edge_pair(G, carry):
            edge_round(2 * G, 0, RING)
            edge_round(2 * G + 1, RING, 0)
            return carry

        lax.fori_loop(0, EDGE_ROUNDS // 2, edge_pair, 0)
        for b in range(RING):
            pltpu.make_async_copy(bufs.at[b], dst.at[idx_v.at[hl + b, 1]],
                                  ssems[b]).wait()
        plsc.subcore_barrier()

        def load(k):
            p = k % 2
            return [pltpu.async_copy(dst.at[lchunk(k)], bufs.at[p],
                                     gsems[p]),
                    pltpu.async_copy(deg_sh.at[lchunk(k)], deg_v.at[p],
                                     isems[p])]

        loads = {0: load(0)}
        o2load = pltpu.async_copy(o2cat.at[gchunk(0)], bufs.at[2], gsems[2])
        writes = {}
        zeros = []
        for k in range(ROW_CHUNKS):
            p = k % 2
            if k + 1 < ROW_CHUNKS:
                if k >= 1:
                    writes.pop(k - 1).wait()
                loads[k + 1] = load(k + 1)
            for d in loads.pop(k):
                d.wait()
            o2load.wait()

            def rbody(r, carry):
                wv = deg_v[p, r, :]
                for f in range(HALF // 16):
                    sf = pl.ds(f * 16, 16)
                    bufs[p, r, sf] = wv * bufs[p, r, sf] + bufs[2, r, sf]
                return carry

            lax.fori_loop(0, CHUNK, rbody, 0)
            if k + 1 < ROW_CHUNKS:
                o2load = pltpu.async_copy(o2cat.at[gchunk(k + 1)],
                                          bufs.at[2], gsems[2])
            writes[k] = pltpu.async_copy(bufs.at[p], dst.at[lchunk(k)],
                                         ssems[p])
            zeros.append(pltpu.async_copy(zero_v, src.at[lchunk(k)],
                                          zsems[k % 2]))
        for d in list(writes.values()) + zeros:
            d.wait()
        plsc.subcore_barrier()

    def power_pair(_, carry):
        one_iter(t0_sh, t1_sh)
        one_iter(t1_sh, t0_sh)
        return carry

    lax.fori_loop(0, N_POWERS // 2, power_pair, 0)

    st = {}
    for k in range(ROW_CHUNKS):
        b = k % RING
        if k >= RING:
            st.pop(k - RING).wait()
        ld = pltpu.async_copy(t0_sh.at[lchunk(k)], bufs.at[b], gsems[b])
        ld.wait()
        st[k] = pltpu.async_copy(bufs.at[b], res.at[gchunk(k)], ssems[b])
    for d in st.values():
        d.wait()


def _propagate(idxm, invcat, o2cat):
    ones_h = jnp.ones((CHUNK, 16), jnp.float32)
    zeros_h = jnp.zeros((CHUNK, HALF), jnp.float32)
    mesh = plsc.VectorSubcoreMesh(core_axis_name="c", subcore_axis_name="s")
    kern = pl.kernel(
        _sc_body,
        out_type=jax.ShapeDtypeStruct((NC * NPAD, HALF), jnp.float32),
        mesh=mesh,
        scratch_types=[
            pltpu.VMEM((2 * RING, 2, CHUNK), jnp.int32),
            pltpu.VMEM((RING, CHUNK, HALF), jnp.float32),
            pltpu.VMEM((CHUNK, HALF), jnp.float32),
            pltpu.VMEM((2, CHUNK, 16), jnp.float32),
            pltpu.VMEM_SHARED((NPAD, HALF), jnp.float32),
            pltpu.VMEM_SHARED((NPAD, HALF), jnp.float32),
            pltpu.VMEM_SHARED((NPAD, 16), jnp.float32),
            [pltpu.SemaphoreType.DMA] * (2 * RING),
            [pltpu.SemaphoreType.DMA] * RING,
            [pltpu.SemaphoreType.DMA] * RING,
            [pltpu.SemaphoreType.DMA] * 2,
        ],
        compiler_params=pltpu.CompilerParams(use_tc_tiling_on_sc=False),
    )
    return kern(idxm, invcat, o2cat, ones_h, zeros_h)


def kernel(x, edge_index, W0, b0, W1, b1):
    x_pad = jnp.pad(x, ((0, NPAD - N_NODES), (0, 0)))
    i0, i1, o0, o1 = _mlp(x_pad, W0, b0, W1, b1)
    invcat = jnp.concatenate([i0, i1], axis=0)
    o2cat = jnp.concatenate([o0, o1], axis=0)

    loop = jnp.arange(N_NODES, dtype=jnp.int32)
    row = jnp.concatenate([edge_index[0], loop])
    col = jnp.concatenate([edge_index[1], loop])
    rowm = jnp.pad(row, (0, E_PAD - E_TOT),
                   constant_values=DUMMY_ROW).reshape(
                       N_SLICES, CHUNKS_PER_SLICE, 1, CHUNK)
    colm = jnp.pad(col, (0, E_PAD - E_TOT), constant_values=0).reshape(
        N_SLICES, CHUNKS_PER_SLICE, 1, CHUNK)
    idxm = jnp.concatenate([colm, rowm], axis=2)

    res = _propagate(idxm, invcat, o2cat)
    return jnp.concatenate(
        [res[:N_NODES], res[NPAD:NPAD + N_NODES]], axis=1)


# speedup vs baseline: 19.4621x; 1.7578x over previous
import jax
import jax.numpy as jnp
from jax import lax
from jax.experimental import pallas as pl
from jax.experimental.pallas import tpu as pltpu
from jax.experimental.pallas import tpu_sc as plsc

N_NODES = 10000
NFEAT = 128
HALF = 64
N_POWERS = 10

NC = 2
NS = 16
CHUNK = 128
RING = 3

NPAD = 10240
ROWS_PER_TILE = NPAD // NS
ROW_CHUNKS = ROWS_PER_TILE // CHUNK
DUMMY_ROW = NPAD - 1

E_TOT = 320000 + N_NODES
N_SLICES = NS
EDGE_ROUNDS = 56
CHUNKS_PER_SLICE = EDGE_ROUNDS * RING
E_PAD = N_SLICES * CHUNKS_PER_SLICE * CHUNK


def _mlp_body(x_ref, w0_ref, b0_ref, w1_ref, b1_ref,
              i0_ref, i1_ref, o0_ref, o1_ref):
    h = jnp.maximum(
        jnp.dot(x_ref[...], w0_ref[...], preferred_element_type=jnp.float32)
        + b0_ref[...], 0.0)
    o = (jnp.dot(h, w1_ref[...], preferred_element_type=jnp.float32)
         + b1_ref[...])
    i0_ref[...] = o[:, :HALF].astype(jnp.bfloat16)
    i1_ref[...] = o[:, HALF:].astype(jnp.bfloat16)
    o0_ref[...] = (0.5 * o[:, :HALF]).astype(jnp.bfloat16)
    o1_ref[...] = (0.5 * o[:, HALF:]).astype(jnp.bfloat16)


def _mlp(x_pad, W0, b0, W1, b1):
    blk = 1024
    grid = NPAD // blk
    outs = [jax.ShapeDtypeStruct((NPAD, HALF), jnp.bfloat16)] * 4
    full = lambda i: (0, 0)
    return pl.pallas_call(
        _mlp_body,
        grid=(grid,),
        in_specs=[
            pl.BlockSpec((blk, NFEAT), lambda i: (i, 0)),
            pl.BlockSpec((NFEAT, NFEAT), full),
            pl.BlockSpec((1, NFEAT), full),
            pl.BlockSpec((NFEAT, NFEAT), full),
            pl.BlockSpec((1, NFEAT), full),
        ],
        out_specs=[pl.BlockSpec((blk, HALF), lambda i: (i, 0))] * 4,
        out_shape=outs,
    )(x_pad, W0, b0.reshape(1, NFEAT), W1, b1.reshape(1, NFEAT))


def _sc_body(idxm, invcat, o2cat, ones_h, zeros_h, zeros_bh,
             res,
             idx_v, bufs, zero_v, zf_v, deg_v,
             t0_sh, t1_sh, deg_sh,
             isems, gsems, ssems, zsems):
    c = lax.axis_index("c")
    s = lax.axis_index("s")

    def lchunk(k):
        return pl.ds(s * ROWS_PER_TILE + k * CHUNK, CHUNK)

    def gchunk(k):
        return pl.ds(c * NPAD + s * ROWS_PER_TILE + k * CHUNK, CHUNK)

    dc = [pltpu.async_copy(ones_h, deg_v.at[1], isems[0]),
          pltpu.async_copy(zeros_bh, zero_v, isems[1]),
          pltpu.async_copy(zeros_h, zf_v, isems[2])]
    dc[1].wait()
    dc[2].wait()
    zd = []
    for k in range(ROW_CHUNKS):
        zd.append(pltpu.async_copy(zero_v, t1_sh.at[lchunk(k)], zsems[0]))
        zd.append(pltpu.async_copy(zf_v, deg_sh.at[lchunk(k)], zsems[1]))
    sd = {}
    for k in range(ROW_CHUNKS):
        b = k % RING
        if k >= RING:
            sd.pop(k - RING).wait()
        ld = pltpu.async_copy(invcat.at[gchunk(k)], bufs.at[b], gsems[b])
        ld.wait()
        sd[k] = pltpu.async_copy(bufs.at[b], t0_sh.at[lchunk(k)], ssems[b])
    dc[0].wait()
    for d in zd + list(sd.values()):
        d.wait()
    plsc.subcore_barrier()

    for b in range(RING):
        pltpu.async_copy(idxm.at[s, b], idx_v.at[b], isems[b])

    def deg_round(g, h, hn):
        for b in range(RING):
            @pl.when(g > 0)
            def _(b=b):
                pltpu.make_async_copy(deg_v.at[1],
                                      deg_sh.at[idx_v.at[hn + b, 1]],
                                      ssems[b]).wait()
            @pl.when(g + 1 < EDGE_ROUNDS)
            def _(b=b):
                pltpu.async_copy(idxm.at[s, (g + 1) * RING + b],
                                 idx_v.at[hn + b], isems[hn + b])
        for b in range(RING):
            pltpu.make_async_copy(idxm.at[s, g * RING + b],
                                  idx_v.at[h + b], isems[h + b]).wait()
            pltpu.async_copy(deg_v.at[1], deg_sh.at[idx_v.at[h + b, 1]],
                             ssems[b], add=True)

    def deg_pair(G, carry):
        deg_round(2 * G, 0, RING)
        deg_round(2 * G + 1, RING, 0)
        return carry

    lax.fori_loop(0, EDGE_ROUNDS // 2, deg_pair, 0)
    hl = ((EDGE_ROUNDS - 1) % 2) * RING
    for b in range(RING):
        pltpu.make_async_copy(deg_v.at[1], deg_sh.at[idx_v.at[hl + b, 1]],
                              ssems[b]).wait()
    plsc.subcore_barrier()

    for k in range(ROW_CHUNKS):
        pltpu.sync_copy(deg_sh.at[lchunk(k)], deg_v.at[0])

        def wbody(r, carry):
            deg_v[0, r, :] = 0.5 / deg_v[0, r, :]
            return carry

        lax.fori_loop(0, CHUNK, wbody, 0)
        pltpu.sync_copy(deg_v.at[0], deg_sh.at[lchunk(k)])
    plsc.subcore_barrier()

    def one_iter(src, dst):
        for b in range(RING):
            pltpu.async_copy(idxm.at[s, b], idx_v.at[b], isems[b])

        def edge_round(g, h, hn):
            for b in range(RING):
                @pl.when(g > 0)
                def _(b=b):
                    pltpu.make_async_copy(
                        bufs.at[b], dst.at[idx_v.at[hn + b, 1]],
                        ssems[b]).wait()
                @pl.when(g + 1 < EDGE_ROUNDS)
                def _(b=b):
                    pltpu.async_copy(idxm.at[s, (g + 1) * RING + b],
                                     idx_v.at[hn + b], isems[hn + b])
            for b in range(RING):
                pltpu.make_async_copy(idxm.at[s, g * RING + b],
                                      idx_v.at[h + b], isems[h + b]).wait()
                pltpu.async_copy(src.at[idx_v.at[h + b, 0]], bufs.at[b],
                                 gsems[b])
            for b in range(RING):
                pltpu.make_async_copy(src.at[idx_v.at[h + b, 0]], bufs.at[b],
                                      gsems[b]).wait()
                pltpu.async_copy(bufs.at[b], dst.at[idx_v.at[h + b, 1]],
                                 ssems[b], add=True)

        def edge_pair(G, carry):
            edge_round(2 * G, 0, RING)
            edge_round(2 * G + 1, RING, 0)
            return carry

        lax.fori_loop(0, EDGE_ROUNDS // 2, edge_pair, 0)
        for b in range(RING):
            pltpu.make_async_copy(bufs.at[b], dst.at[idx_v.at[hl + b, 1]],
                                  ssems[b]).wait()
        plsc.subcore_barrier()

        def load(k):
            p = k % 2
            return [pltpu.async_copy(dst.at[lchunk(k)], bufs.at[p],
                                     gsems[p]),
                    pltpu.async_copy(deg_sh.at[lchunk(k)], deg_v.at[p],
                                     isems[p])]

        loads = {0: load(0)}
        o2load = pltpu.async_copy(o2cat.at[gchunk(0)], bufs.at[2], gsems[2])
        writes = {}
        zeros = []
        for k in range(ROW_CHUNKS):
            p = k % 2
            if k + 1 < ROW_CHUNKS:
                if k >= 1:
                    writes.pop(k - 1).wait()
                loads[k + 1] = load(k + 1)
            for d in loads.pop(k):
                d.wait()
            o2load.wait()

            def rbody(r, carry):
                wv = deg_v[p, r, :]
                for f in range(HALF // 16):
                    sf = pl.ds(f * 16, 16)
                    bufs[p, r, sf] = (
                        wv * bufs[p, r, sf].astype(jnp.float32)
                        + bufs[2, r, sf].astype(jnp.float32)
                    ).astype(jnp.bfloat16)
                return carry

            lax.fori_loop(0, CHUNK, rbody, 0)
            if k + 1 < ROW_CHUNKS:
                o2load = pltpu.async_copy(o2cat.at[gchunk(k + 1)],
                                          bufs.at[2], gsems[2])
            writes[k] = pltpu.async_copy(bufs.at[p], dst.at[lchunk(k)],
                                         ssems[p])
            zeros.append(pltpu.async_copy(zero_v, src.at[lchunk(k)],
                                          zsems[k % 2]))
        for d in list(writes.values()) + zeros:
            d.wait()
        plsc.subcore_barrier()

    def power_pair(_, carry):
        one_iter(t0_sh, t1_sh)
        one_iter(t1_sh, t0_sh)
        return carry

    lax.fori_loop(0, N_POWERS // 2, power_pair, 0)

    st = {}
    for k in range(ROW_CHUNKS):
        b = k % RING
        if k >= RING:
            st.pop(k - RING).wait()
        ld = pltpu.async_copy(t0_sh.at[lchunk(k)], bufs.at[b], gsems[b])
        ld.wait()
        st[k] = pltpu.async_copy(bufs.at[b], res.at[gchunk(k)], ssems[b])
    for d in st.values():
        d.wait()


def _propagate(idxm, invcat, o2cat):
    ones_h = jnp.ones((CHUNK, 16), jnp.float32)
    zeros_h = jnp.zeros((CHUNK, 16), jnp.float32)
    zeros_bh = jnp.zeros((CHUNK, HALF), jnp.bfloat16)
    mesh = plsc.VectorSubcoreMesh(core_axis_name="c", subcore_axis_name="s")
    kern = pl.kernel(
        _sc_body,
        out_type=jax.ShapeDtypeStruct((NC * NPAD, HALF), jnp.bfloat16),
        mesh=mesh,
        scratch_types=[
            pltpu.VMEM((2 * RING, 2, CHUNK), jnp.int32),
            pltpu.VMEM((RING, CHUNK, HALF), jnp.bfloat16),
            pltpu.VMEM((CHUNK, HALF), jnp.bfloat16),
            pltpu.VMEM((CHUNK, 16), jnp.float32),
            pltpu.VMEM((2, CHUNK, 16), jnp.float32),
            pltpu.VMEM_SHARED((NPAD, HALF), jnp.bfloat16),
            pltpu.VMEM_SHARED((NPAD, HALF), jnp.bfloat16),
            pltpu.VMEM_SHARED((NPAD, 16), jnp.float32),
            [pltpu.SemaphoreType.DMA] * (2 * RING),
            [pltpu.SemaphoreType.DMA] * RING,
            [pltpu.SemaphoreType.DMA] * RING,
            [pltpu.SemaphoreType.DMA] * 2,
        ],
        compiler_params=pltpu.CompilerParams(use_tc_tiling_on_sc=False),
    )
    return kern(idxm, invcat, o2cat, ones_h, zeros_h, zeros_bh)


def kernel(x, edge_index, W0, b0, W1, b1):
    x_pad = jnp.pad(x, ((0, NPAD - N_NODES), (0, 0)))
    i0, i1, o0, o1 = _mlp(x_pad, W0, b0, W1, b1)
    invcat = jnp.concatenate([i0, i1], axis=0)
    o2cat = jnp.concatenate([o0, o1], axis=0)

    loop = jnp.arange(N_NODES, dtype=jnp.int32)
    row = jnp.concatenate([edge_index[0], loop])
    col = jnp.concatenate([edge_index[1], loop])
    rowm = jnp.pad(row, (0, E_PAD - E_TOT),
                   constant_values=DUMMY_ROW).reshape(
                       N_SLICES, CHUNKS_PER_SLICE, 1, CHUNK)
    colm = jnp.pad(col, (0, E_PAD - E_TOT), constant_values=0).reshape(
        N_SLICES, CHUNKS_PER_SLICE, 1, CHUNK)
    idxm = jnp.concatenate([colm, rowm], axis=2)

    res = _propagate(idxm, invcat, o2cat).astype(jnp.float32)
    return jnp.concatenate(
        [res[:N_NODES], res[NPAD:NPAD + N_NODES]], axis=1)


# speedup vs baseline: 24.8649x; 1.2776x over previous
import jax
import jax.numpy as jnp
from jax import lax
from jax.experimental import pallas as pl
from jax.experimental.pallas import tpu as pltpu
from jax.experimental.pallas import tpu_sc as plsc

N_NODES = 10000
NFEAT = 128
HALF = 64
N_POWERS = 10

NC = 2
NS = 16
CHUNK = 128
RING = 3

NPAD = 10240
ROWS_PER_TILE = NPAD // NS
ROW_CHUNKS = ROWS_PER_TILE // CHUNK
DUMMY_ROW = NPAD - 1

E_TOT = 320000 + N_NODES
N_SLICES = NS
EDGE_ROUNDS = 54
CHUNKS_PER_SLICE = EDGE_ROUNDS * RING
E_PAD = N_SLICES * CHUNKS_PER_SLICE * CHUNK


def _mlp_body(x_ref, w0_ref, b0_ref, w1_ref, b1_ref,
              i0_ref, i1_ref, o0_ref, o1_ref):
    h = jnp.maximum(
        jnp.dot(x_ref[...], w0_ref[...], preferred_element_type=jnp.float32)
        + b0_ref[...], 0.0)
    o = (jnp.dot(h, w1_ref[...], preferred_element_type=jnp.float32)
         + b1_ref[...])
    i0_ref[...] = o[:, :HALF].astype(jnp.bfloat16)
    i1_ref[...] = o[:, HALF:].astype(jnp.bfloat16)
    o0_ref[...] = (0.5 * o[:, :HALF]).astype(jnp.bfloat16)
    o1_ref[...] = (0.5 * o[:, HALF:]).astype(jnp.bfloat16)


def _mlp(x_pad, W0, b0, W1, b1):
    blk = 1024
    grid = NPAD // blk
    outs = [jax.ShapeDtypeStruct((NPAD, HALF), jnp.bfloat16)] * 4
    full = lambda i: (0, 0)
    return pl.pallas_call(
        _mlp_body,
        grid=(grid,),
        in_specs=[
            pl.BlockSpec((blk, NFEAT), lambda i: (i, 0)),
            pl.BlockSpec((NFEAT, NFEAT), full),
            pl.BlockSpec((1, NFEAT), full),
            pl.BlockSpec((NFEAT, NFEAT), full),
            pl.BlockSpec((1, NFEAT), full),
        ],
        out_specs=[pl.BlockSpec((blk, HALF), lambda i: (i, 0))] * 4,
        out_shape=outs,
    )(x_pad, W0, b0.reshape(1, NFEAT), W1, b1.reshape(1, NFEAT))


def _sc_body(idxm, invcat, o2cat, ones_h, zeros_h, zeros_bh,
             res,
             idx_v, bufs, zero_v, zf_v, deg_v,
             t0_sh, t1_sh, deg_sh,
             isems, gsems, ssems, zsems):
    c = lax.axis_index("c")
    s = lax.axis_index("s")

    def lchunk(k):
        return pl.ds(s * ROWS_PER_TILE + k * CHUNK, CHUNK)

    def gchunk(k):
        return pl.ds(c * NPAD + s * ROWS_PER_TILE + k * CHUNK, CHUNK)

    dc = [pltpu.async_copy(ones_h, deg_v.at[1], isems[0]),
          pltpu.async_copy(zeros_bh, zero_v, isems[1]),
          pltpu.async_copy(zeros_h, zf_v, isems[2])]
    dc[1].wait()
    dc[2].wait()
    zd = []
    for k in range(ROW_CHUNKS):
        zd.append(pltpu.async_copy(zero_v, t1_sh.at[lchunk(k)], zsems[0]))
        zd.append(pltpu.async_copy(zf_v, deg_sh.at[lchunk(k)], zsems[1]))
    sd = {}
    for k in range(ROW_CHUNKS):
        b = k % RING
        if k >= RING:
            sd.pop(k - RING).wait()
        ld = pltpu.async_copy(invcat.at[gchunk(k)], bufs.at[b], gsems[b])
        ld.wait()
        sd[k] = pltpu.async_copy(bufs.at[b], t0_sh.at[lchunk(k)], ssems[b])
    dc[0].wait()
    for d in zd + list(sd.values()):
        d.wait()
    plsc.subcore_barrier()

    for b in range(RING):
        pltpu.async_copy(idxm.at[s, b], idx_v.at[b], isems[b])

    def deg_round(g, h, hn):
        for b in range(RING):
            @pl.when(g > 0)
            def _(b=b):
                pltpu.make_async_copy(deg_v.at[1],
                                      deg_sh.at[idx_v.at[hn + b, 1]],
                                      ssems[b]).wait()
            @pl.when(g + 1 < EDGE_ROUNDS)
            def _(b=b):
                pltpu.async_copy(idxm.at[s, (g + 1) * RING + b],
                                 idx_v.at[hn + b], isems[hn + b])
        for b in range(RING):
            pltpu.make_async_copy(idxm.at[s, g * RING + b],
                                  idx_v.at[h + b], isems[h + b]).wait()
            pltpu.async_copy(deg_v.at[1], deg_sh.at[idx_v.at[h + b, 1]],
                             ssems[b], add=True)

    def deg_pair(G, carry):
        deg_round(2 * G, 0, RING)
        deg_round(2 * G + 1, RING, 0)
        return carry

    lax.fori_loop(0, EDGE_ROUNDS // 2, deg_pair, 0)
    hl = ((EDGE_ROUNDS - 1) % 2) * RING
    for b in range(RING):
        pltpu.make_async_copy(deg_v.at[1], deg_sh.at[idx_v.at[hl + b, 1]],
                              ssems[b]).wait()
    plsc.subcore_barrier()

    for k in range(ROW_CHUNKS):
        pltpu.sync_copy(deg_sh.at[lchunk(k)], deg_v.at[0])

        def wbody(r, carry):
            deg_v[0, r, :] = 0.5 / deg_v[0, r, :]
            return carry

        lax.fori_loop(0, CHUNK, wbody, 0)
        pltpu.sync_copy(deg_v.at[0], deg_sh.at[lchunk(k)])
    plsc.subcore_barrier()

    def one_iter(src, dst, final=False):
        for b in range(RING):
            pltpu.async_copy(idxm.at[s, b], idx_v.at[b], isems[b])

        def edge_round(g, h, hn):
            for b in range(RING):
                @pl.when(g > 0)
                def _(b=b):
                    pltpu.make_async_copy(
                        bufs.at[b], dst.at[idx_v.at[hn + b, 1]],
                        ssems[b]).wait()
                @pl.when(g + 1 < EDGE_ROUNDS)
                def _(b=b):
                    pltpu.async_copy(idxm.at[s, (g + 1) * RING + b],
                                     idx_v.at[hn + b], isems[hn + b])
            for b in range(RING):
                pltpu.make_async_copy(idxm.at[s, g * RING + b],
                                      idx_v.at[h + b], isems[h + b]).wait()
                pltpu.async_copy(src.at[idx_v.at[h + b, 0]], bufs.at[b],
                                 gsems[b])
            for b in range(RING):
                pltpu.make_async_copy(src.at[idx_v.at[h + b, 0]], bufs.at[b],
                                      gsems[b]).wait()
                pltpu.async_copy(bufs.at[b], dst.at[idx_v.at[h + b, 1]],
                                 ssems[b], add=True)

        def edge_pair(G, carry):
            edge_round(2 * G, 0, RING)
            edge_round(2 * G + 1, RING, 0)
            return carry

        lax.fori_loop(0, EDGE_ROUNDS // 2, edge_pair, 0)
        for b in range(RING):
            pltpu.make_async_copy(bufs.at[b], dst.at[idx_v.at[hl + b, 1]],
                                  ssems[b]).wait()
        plsc.subcore_barrier()

        def load(k):
            p = k % 2
            return [pltpu.async_copy(dst.at[lchunk(k)], bufs.at[p],
                                     gsems[p]),
                    pltpu.async_copy(deg_sh.at[lchunk(k)], deg_v.at[p],
                                     isems[p])]

        loads = {0: load(0)}
        o2load = pltpu.async_copy(o2cat.at[gchunk(0)], bufs.at[2], gsems[2])
        writes = {}
        writes2 = {}
        zeros = []
        for k in range(ROW_CHUNKS):
            p = k % 2
            if k + 1 < ROW_CHUNKS:
                if k >= 1:
                    writes.pop(k - 1).wait()
                    if final:
                        writes2.pop(k - 1).wait()
                loads[k + 1] = load(k + 1)
            for d in loads.pop(k):
                d.wait()
            o2load.wait()

            def rbody(r, carry):
                wv = deg_v[p, r, :]
                for f in range(HALF // 16):
                    sf = pl.ds(f * 16, 16)
                    bufs[p, r, sf] = (
                        wv * bufs[p, r, sf].astype(jnp.float32)
                        + bufs[2, r, sf].astype(jnp.float32)
                    ).astype(jnp.bfloat16)
                return carry

            lax.fori_loop(0, CHUNK, rbody, 0)
            if k + 1 < ROW_CHUNKS:
                o2load = pltpu.async_copy(o2cat.at[gchunk(k + 1)],
                                          bufs.at[2], gsems[2])
            writes[k] = pltpu.async_copy(bufs.at[p], dst.at[lchunk(k)],
                                         ssems[p])
            if final:
                writes2[k] = pltpu.async_copy(bufs.at[p], res.at[gchunk(k)],
                                              isems[2 + p])
            else:
                zeros.append(pltpu.async_copy(zero_v, src.at[lchunk(k)],
                                              zsems[k % 2]))
        for d in list(writes.values()) + list(writes2.values()) + zeros:
            d.wait()
        plsc.subcore_barrier()

    def power_pair(_, carry):
        one_iter(t0_sh, t1_sh)
        one_iter(t1_sh, t0_sh)
        return carry

    lax.fori_loop(0, N_POWERS // 2 - 1, power_pair, 0)
    one_iter(t0_sh, t1_sh)
    one_iter(t1_sh, t0_sh, final=True)


def _propagate(idxm, invcat, o2cat):
    ones_h = jnp.ones((CHUNK, 16), jnp.float32)
    zeros_h = jnp.zeros((CHUNK, 16), jnp.float32)
    zeros_bh = jnp.zeros((CHUNK, HALF), jnp.bfloat16)
    mesh = plsc.VectorSubcoreMesh(core_axis_name="c", subcore_axis_name="s")
    kern = pl.kernel(
        _sc_body,
        out_type=jax.ShapeDtypeStruct((NC * NPAD, HALF), jnp.bfloat16),
        mesh=mesh,
        scratch_types=[
            pltpu.VMEM((2 * RING, 2, CHUNK), jnp.int32),
            pltpu.VMEM((RING, CHUNK, HALF), jnp.bfloat16),
            pltpu.VMEM((CHUNK, HALF), jnp.bfloat16),
            pltpu.VMEM((CHUNK, 16), jnp.float32),
            pltpu.VMEM((2, CHUNK, 16), jnp.float32),
            pltpu.VMEM_SHARED((NPAD, HALF), jnp.bfloat16),
            pltpu.VMEM_SHARED((NPAD, HALF), jnp.bfloat16),
            pltpu.VMEM_SHARED((NPAD, 16), jnp.float32),
            [pltpu.SemaphoreType.DMA] * (2 * RING),
            [pltpu.SemaphoreType.DMA] * RING,
            [pltpu.SemaphoreType.DMA] * RING,
            [pltpu.SemaphoreType.DMA] * 2,
        ],
        compiler_params=pltpu.CompilerParams(use_tc_tiling_on_sc=False),
    )
    return kern(idxm, invcat, o2cat, ones_h, zeros_h, zeros_bh)


def kernel(x, edge_index, W0, b0, W1, b1):
    x_pad = jnp.pad(x, ((0, NPAD - N_NODES), (0, 0)))
    i0, i1, o0, o1 = _mlp(x_pad, W0, b0, W1, b1)
    invcat = jnp.concatenate([i0, i1], axis=0)
    o2cat = jnp.concatenate([o0, o1], axis=0)

    loop = jnp.arange(N_NODES, dtype=jnp.int32)
    row = jnp.concatenate([edge_index[0], loop])
    col = jnp.concatenate([edge_index[1], loop])
    rowm = jnp.pad(row, (0, E_PAD - E_TOT),
                   constant_values=DUMMY_ROW).reshape(
                       N_SLICES, CHUNKS_PER_SLICE, 1, CHUNK)
    colm = jnp.pad(col, (0, E_PAD - E_TOT), constant_values=0).reshape(
        N_SLICES, CHUNKS_PER_SLICE, 1, CHUNK)
    idxm = jnp.concatenate([colm, rowm], axis=2)

    res = _propagate(idxm, invcat, o2cat).astype(jnp.float32)
    return jnp.concatenate(
        [res[:N_NODES], res[NPAD:NPAD + N_NODES]], axis=1)
